# baseline (speedup 1.0000x reference)
import functools
import math

import jax
import jax.numpy as jnp
import numpy as np
from jax.experimental import pallas as pl
from jax.experimental.pallas import tpu as pltpu

_VMEM_LIMIT_BYTES = 48 * 1024 * 1024


def _interp_matrix(out_size: int, in_size: int) -> np.ndarray:
    if out_size == 1 or in_size == 1:
        src = np.zeros((out_size,), dtype=np.float64)
    else:
        src = np.arange(out_size, dtype=np.float64) * (in_size - 1) / (out_size - 1)
    i0 = np.clip(np.floor(src).astype(np.int64), 0, in_size - 1)
    i1 = np.clip(i0 + 1, 0, in_size - 1)
    frac = src - i0
    a = np.zeros((out_size, in_size), dtype=np.float64)
    a[np.arange(out_size), i0] += 1.0 - frac
    a[np.arange(out_size), i1] += frac
    return a


def _kron_interp_matrix(h_out, h_in, w_out, w_in) -> np.ndarray:
    ah = _interp_matrix(h_out, h_in)
    aw = _interp_matrix(w_out, w_in)
    m = np.einsum("Hh,Ww->hwHW", ah, aw)
    return m.reshape(h_in * w_in, h_out * w_out)


def _largest_divisor_leq(n: int, cap: int) -> int:
    cap = max(1, min(n, cap))
    for d in range(cap, 0, -1):
        if n % d == 0:
            return d
    return 1


def _fused_kernel(x_ref, skip_ref, m_ref, o_ref, *, jx):
    j = pl.program_id(1)

    @pl.when(j < jx)
    def _upsample():
        tb, h_in, w_in = x_ref.shape
        _, h_out, w_out = o_ref.shape
        xb = x_ref[...].reshape(tb, h_in * w_in).astype(jnp.bfloat16)
        acc = jnp.dot(xb, m_ref[...], preferred_element_type=jnp.float32)
        o_ref[...] = acc.reshape(tb, h_out, w_out)

    @pl.when(j >= jx)
    def _copy_skip():
        o_ref[...] = skip_ref[...]


def _upsample_only_kernel(x_ref, m_ref, o_ref):
    tb, h_in, w_in = x_ref.shape
    _, h_out, w_out = o_ref.shape
    xb = x_ref[...].reshape(tb, h_in * w_in).astype(jnp.bfloat16)
    acc = jnp.dot(xb, m_ref[...], preferred_element_type=jnp.float32)
    o_ref[...] = acc.reshape(tb, h_out, w_out)


def _upsample_align_corners(x, h_out, w_out):
    n, c, h_in, w_in = x.shape
    rows = n * c
    m = jnp.asarray(_kron_interp_matrix(h_out, h_in, w_out, w_in), dtype=jnp.bfloat16)
    tb = _largest_divisor_leq(rows, 128)
    x_flat = x.reshape(rows, h_in, w_in)
    out_flat = pl.pallas_call(
        _upsample_only_kernel,
        out_shape=jax.ShapeDtypeStruct((rows, h_out, w_out), x.dtype),
        grid=(rows // tb,),
        in_specs=[
            pl.BlockSpec((tb, h_in, w_in), lambda i: (i, 0, 0)),
            pl.BlockSpec((h_in * w_in, h_out * w_out), lambda i: (0, 0)),
        ],
        out_specs=pl.BlockSpec((tb, h_out, w_out), lambda i: (i, 0, 0)),
        compiler_params=pltpu.CompilerParams(
            dimension_semantics=("parallel",),
            vmem_limit_bytes=_VMEM_LIMIT_BYTES),
    )(x_flat, m)
    return out_flat.reshape(n, c, h_out, w_out)


def kernel(x, skip):
    n, c_x, h_in, w_in = x.shape
    n2, c_s, h_out, w_out = skip.shape
    assert n == n2, (x.shape, skip.shape)

    tb = _largest_divisor_leq(math.gcd(c_x, c_s), 128)
    if tb < 2 or (h_in * w_in) % 8 != 0:
        up = _upsample_align_corners(x, h_out, w_out)
        return jnp.concatenate([up, skip], axis=1)

    jx = c_x // tb
    js = c_s // tb
    jt = jx + js
    c_total = c_x + c_s

    m = jnp.asarray(_kron_interp_matrix(h_out, h_in, w_out, w_in), dtype=jnp.bfloat16)

    x_flat = x.reshape(n * c_x, h_in, w_in)
    skip_flat = skip.reshape(n * c_s, h_out, w_out)

    def x_map(nn, j):
        return (nn * jx + jnp.minimum(j, jx - 1), 0, 0)

    def skip_map(nn, j):
        return (nn * js + jnp.maximum(j - jx, 0), 0, 0)

    def m_map(nn, j):
        return (0, 0)

    def out_map(nn, j):
        return (nn * jt + j, 0, 0)

    out_flat = pl.pallas_call(
        functools.partial(_fused_kernel, jx=jx),
        out_shape=jax.ShapeDtypeStruct((n * c_total, h_out, w_out), x.dtype),
        grid=(n, jt),
        in_specs=[
            pl.BlockSpec((tb, h_in, w_in), x_map),
            pl.BlockSpec((tb, h_out, w_out), skip_map),
            pl.BlockSpec((h_in * w_in, h_out * w_out), m_map),
        ],
        out_specs=pl.BlockSpec((tb, h_out, w_out), out_map),
        compiler_params=pltpu.CompilerParams(
            dimension_semantics=("parallel", "parallel"),
            vmem_limit_bytes=_VMEM_LIMIT_BYTES),
    )(x_flat, skip_flat, m)

    return out_flat.reshape(n, c_total, h_out, w_out)
